# Initial kernel scaffold; baseline (speedup 1.0000x reference)
import jax
import jax.numpy as jnp
from jax.experimental import pallas as pl


def kernel(x, edge_index, edge_weight, W_self_0, W_neigh_0, b_0, W_self_1, W_neigh_1, b_1):
    raise NotImplementedError("write your pallas kernel here")


# speedup vs baseline: 2.3844x; 2.3844x over previous
import functools

import jax
import jax.numpy as jnp
from jax import lax
from jax.experimental import pallas as pl
from jax.experimental.pallas import tpu as pltpu
from jax.experimental.pallas import tpu_sc as plsc

N = 10000
E = 160000
D = 256
HALF = 128
NC = 2
NS = 16
LANES = 16
CH = 64
G = 10
NGRP = 16
NCHUNK = NGRP * G
EPT = NCHUNK * CH
EPAD = NS * EPT
NACC = 10240
APT = NACC // NS
CCH = EPAD // NC // NS // CH


def _sc_agg_body(tab_h, col_h, row_h, ew_h, z_h, agg_h,
                 col_s, row_s, ew_s, rows_v, acc, semg):
  c = lax.axis_index("c")
  s = lax.axis_index("s")

  pltpu.sync_copy(z_h, acc.at[pl.ds(s * APT, APT)])

  def scale(k):
    off = k * CH

    def grp(i, carry):
      ew16 = ew_s[pl.ds(off + i * LANES, LANES)]
      for e in range(LANES):
        wv = jnp.full((LANES,), ew16[e], jnp.float32)
        ri = i * LANES + e
        for f in range(HALF // LANES):
          sl = pl.ds(f * LANES, LANES)
          rows_v[ri, sl] = rows_v[ri, sl] * wv
      return carry

    lax.fori_loop(0, CH // LANES, grp, 0)

  plsc.subcore_barrier()

  def group_body(g, carry):
    pltpu.sync_copy(col_h.at[c, s, g], col_s)
    pltpu.sync_copy(row_h.at[s, g], row_s)
    pltpu.sync_copy(ew_h.at[s, g], ew_s)
    for k in range(G):
      pltpu.async_copy(tab_h.at[col_s.at[k]], rows_v, semg).wait()
      scale(k)
      pltpu.sync_copy(rows_v, acc.at[row_s.at[k]], add=True)
    return carry

  lax.fori_loop(0, NGRP, group_body, 0)

  plsc.subcore_barrier()
  pltpu.sync_copy(acc.at[pl.ds(s * APT, APT)], agg_h.at[c, s])


def _sc_aggregate(tab, col2, row_r, ew_r, z):
  mesh = plsc.VectorSubcoreMesh(core_axis_name="c", subcore_axis_name="s",
                                num_cores=NC, num_subcores=NS)
  fn = pl.kernel(
      _sc_agg_body,
      out_type=jax.ShapeDtypeStruct((NC, NS, APT, HALF), jnp.float32),
      mesh=mesh,
      scratch_types=[
          pltpu.VMEM((G, CH), jnp.int32),
          pltpu.VMEM((G, CH), jnp.int32),
          pltpu.VMEM((G * CH,), jnp.float32),
          pltpu.VMEM((CH, HALF), jnp.float32),
          pltpu.VMEM_SHARED((NACC, HALF), jnp.float32),
          pltpu.SemaphoreType.DMA,
      ],
  )
  return fn(tab, col2, row_r, ew_r, z)


def _sc_cnt_body(rowc_h, z_h, ones_h, cnt_h, row_s, ones_v, acc, sem):
  c = lax.axis_index("c")
  s = lax.axis_index("s")
  pltpu.sync_copy(rowc_h.at[c, s], row_s)
  pltpu.sync_copy(ones_h, ones_v)
  pltpu.sync_copy(z_h, acc.at[pl.ds(s * APT, APT)])
  plsc.subcore_barrier()

  def body(j, carry):
    pltpu.sync_copy(ones_v, acc.at[row_s.at[j]], add=True)
    return carry

  lax.fori_loop(0, CCH, body, 0)
  plsc.subcore_barrier()
  pltpu.sync_copy(acc.at[pl.ds(s * APT, APT)], cnt_h.at[c, s])


def _sc_count(rowc, z, ones128):
  mesh = plsc.VectorSubcoreMesh(core_axis_name="c", subcore_axis_name="s",
                                num_cores=NC, num_subcores=NS)
  fn = pl.kernel(
      _sc_cnt_body,
      out_type=jax.ShapeDtypeStruct((NC, NS, APT, HALF), jnp.float32),
      mesh=mesh,
      scratch_types=[
          pltpu.VMEM((CCH, CH), jnp.int32),
          pltpu.VMEM((CH, HALF), jnp.float32),
          pltpu.VMEM_SHARED((NACC, HALF), jnp.float32),
          pltpu.SemaphoreType.DMA,
      ],
  )
  return fn(rowc, z, ones128)


def _tc_body(relu, stacked, h2_ref, agg2_ref, cnt_ref, ws_ref, wn_ref,
             b_ref, o_ref):
  cnt = cnt_ref[0][:, 0:1] + cnt_ref[1][:, 0:1]
  inv = 1.0 / jnp.maximum(cnt, 1.0)
  dn = (((1,), (0,)), ((), ()))
  r = lax.dot_general(h2_ref[0], ws_ref[0:HALF, :], dn,
                      preferred_element_type=jnp.float32)
  r = r + lax.dot_general(h2_ref[1], ws_ref[HALF:, :], dn,
                          preferred_element_type=jnp.float32)
  r = r + lax.dot_general(agg2_ref[0] * inv, wn_ref[0:HALF, :], dn,
                          preferred_element_type=jnp.float32)
  r = r + lax.dot_general(agg2_ref[1] * inv, wn_ref[HALF:, :], dn,
                          preferred_element_type=jnp.float32)
  r = r + b_ref[...]
  if relu:
    r = jnp.maximum(r, 0.0)
  if stacked:
    o_ref[0] = r[:, :HALF]
    o_ref[1] = r[:, HALF:]
  else:
    o_ref[...] = r


def _tc_layer(h2, aggp, cntp, W_self, W_neigh, bias, relu, stacked):
  Bm = 2000
  grid = (N // Bm,)
  if stacked:
    out_shape = jax.ShapeDtypeStruct((NC, N, HALF), jnp.float32)
    out_spec = pl.BlockSpec((NC, Bm, HALF), lambda m: (0, m, 0))
  else:
    out_shape = jax.ShapeDtypeStruct((N, D), jnp.float32)
    out_spec = pl.BlockSpec((Bm, D), lambda m: (m, 0))
  return pl.pallas_call(
      functools.partial(_tc_body, relu, stacked),
      grid=grid,
      in_specs=[
          pl.BlockSpec((NC, Bm, HALF), lambda m: (0, m, 0)),
          pl.BlockSpec((NC, Bm, HALF), lambda m: (0, m, 0)),
          pl.BlockSpec((NC, Bm, HALF), lambda m: (0, m, 0)),
          pl.BlockSpec((D, D), lambda m: (0, 0)),
          pl.BlockSpec((D, D), lambda m: (0, 0)),
          pl.BlockSpec((1, D), lambda m: (0, 0)),
      ],
      out_specs=out_spec,
      out_shape=out_shape,
  )(h2, aggp, cntp, W_self, W_neigh, bias)


def kernel(x, edge_index, edge_weight, W_self_0, W_neigh_0, b_0,
           W_self_1, W_neigh_1, b_1):
  row = edge_index[0]
  col = edge_index[1]
  pad = EPAD - E
  colp = jnp.concatenate([col, jnp.zeros((pad,), col.dtype)])
  rowp = jnp.concatenate([row, jnp.full((pad,), N, row.dtype)])
  ewp = jnp.concatenate([edge_weight, jnp.zeros((pad,), jnp.float32)])
  x2 = jnp.stack([x[:, :HALF], x[:, HALF:]])
  colr = colp.reshape(NS, NGRP, G, CH)
  col2 = jnp.stack([colr, colr + N])
  row_r = rowp.reshape(NS, NGRP, G, CH)
  ew_r = ewp.reshape(NS, NGRP, G * CH)
  rowc = rowp.reshape(NC, NS, CCH, CH)
  z = jnp.zeros((APT, HALF), jnp.float32)
  ones128 = jnp.ones((CH, HALF), jnp.float32)

  cntp = _sc_count(rowc, z, ones128).reshape(NC, NACC, HALF)
  agg0 = _sc_aggregate(x2.reshape(NC * N, HALF), col2, row_r, ew_r,
                       z).reshape(NC, NACC, HALF)
  h2 = _tc_layer(x2, agg0, cntp, W_self_0, W_neigh_0,
                 b_0.reshape(1, D), relu=True, stacked=True)
  agg1 = _sc_aggregate(h2.reshape(NC * N, HALF), col2, row_r, ew_r,
                       z).reshape(NC, NACC, HALF)
  return _tc_layer(h2, agg1, cntp, W_self_1, W_neigh_1,
                   b_1.reshape(1, D), relu=False, stacked=False)


# speedup vs baseline: 3.3461x; 1.4033x over previous
import functools

import jax
import jax.numpy as jnp
from jax import lax
from jax.experimental import pallas as pl
from jax.experimental.pallas import tpu as pltpu
from jax.experimental.pallas import tpu_sc as plsc

N = 10000
E = 160000
D = 256
HALF = 128
NC = 2
NS = 16
LANES = 16
CH = 64
G = 10
NGRP = 16
NCHUNK = NGRP * G
EPT = NCHUNK * CH
EPAD = NS * EPT
NACC = 10240
APT = NACC // NS
CCH = EPAD // NC // NS // CH


def _sc_agg_body(tab_h, col_h, row_h, ew_h, z_h, agg_h,
                 col_s, row_s, ew_s, rows_v, acc, semg0, semg1, semst):
  c = lax.axis_index("c")
  s = lax.axis_index("s")
  sems = (semg0, semg1)

  pltpu.sync_copy(z_h, acc.at[pl.ds(s * APT, APT)])

  def stage_start(g, gb):
    pltpu.async_copy(col_h.at[c, s, g], col_s.at[gb], semst)
    pltpu.async_copy(row_h.at[s, g], row_s.at[gb], semst)
    pltpu.async_copy(ew_h.at[s, g], ew_s.at[gb], semst)

  def stage_wait(g, gb):
    pltpu.make_async_copy(col_h.at[c, s, g], col_s.at[gb], semst).wait()
    pltpu.make_async_copy(row_h.at[s, g], row_s.at[gb], semst).wait()
    pltpu.make_async_copy(ew_h.at[s, g], ew_s.at[gb], semst).wait()

  def issue_gather(gb, k):
    pltpu.async_copy(tab_h.at[col_s.at[gb, k]], rows_v.at[k % 2],
                     sems[k % 2])

  def wait_gather(gb, k):
    pltpu.make_async_copy(tab_h.at[col_s.at[gb, k]], rows_v.at[k % 2],
                          sems[k % 2]).wait()

  def scale(gb, k):
    b = k % 2
    off = k * CH

    def grp(i, carry):
      ew16 = ew_s[gb, pl.ds(off + i * LANES, LANES)]
      for e in range(LANES):
        wv = jnp.full((LANES,), ew16[e], jnp.float32)
        ri = i * LANES + e
        for f in range(HALF // LANES):
          sl = pl.ds(f * LANES, LANES)
          rows_v[b, ri, sl] = rows_v[b, ri, sl] * wv
      return carry

    lax.fori_loop(0, CH // LANES, grp, 0)

  plsc.subcore_barrier()

  stage_start(0, 0)
  stage_wait(0, 0)
  stage_start(1, 1)
  issue_gather(0, 0)
  issue_gather(0, 1)

  def group_body(g, carry):
    gb = lax.rem(g, 2)
    ngb = lax.rem(g + 1, 2)
    for k in range(G):
      wait_gather(gb, k)
      scale(gb, k)
      pltpu.sync_copy(rows_v.at[k % 2], acc.at[row_s.at[gb, k]], add=True)
      if k <= G - 3:
        issue_gather(gb, k + 2)
      if k == G - 3:
        @pl.when(g + 1 < NGRP)
        def _():
          stage_wait(g + 1, ngb)
      if k == G - 2:
        @pl.when(g + 1 < NGRP)
        def _():
          issue_gather(ngb, 0)
      if k == G - 1:
        @pl.when(g + 1 < NGRP)
        def _():
          issue_gather(ngb, 1)

        @pl.when(g + 2 < NGRP)
        def _():
          stage_start(g + 2, gb)
    return carry

  lax.fori_loop(0, NGRP, group_body, 0)

  plsc.subcore_barrier()
  pltpu.sync_copy(acc.at[pl.ds(s * APT, APT)], agg_h.at[c, s])


def _sc_aggregate(tab, col2, row_r, ew_r, z):
  mesh = plsc.VectorSubcoreMesh(core_axis_name="c", subcore_axis_name="s",
                                num_cores=NC, num_subcores=NS)
  fn = pl.kernel(
      _sc_agg_body,
      out_type=jax.ShapeDtypeStruct((NC, NS, APT, HALF), jnp.float32),
      mesh=mesh,
      scratch_types=[
          pltpu.VMEM((2, G, CH), jnp.int32),
          pltpu.VMEM((2, G, CH), jnp.int32),
          pltpu.VMEM((2, G * CH), jnp.float32),
          pltpu.VMEM((2, CH, HALF), jnp.float32),
          pltpu.VMEM_SHARED((NACC, HALF), jnp.float32),
          pltpu.SemaphoreType.DMA,
          pltpu.SemaphoreType.DMA,
          pltpu.SemaphoreType.DMA,
      ],
  )
  return fn(tab, col2, row_r, ew_r, z)


def _sc_cnt_body(rowc_h, z_h, ones_h, cnt_h, row_s, ones_v, acc, sem):
  c = lax.axis_index("c")
  s = lax.axis_index("s")
  pltpu.sync_copy(rowc_h.at[c, s], row_s)
  pltpu.sync_copy(ones_h, ones_v)
  pltpu.sync_copy(z_h, acc.at[pl.ds(s * APT, APT)])
  plsc.subcore_barrier()

  def body(j, carry):
    pltpu.sync_copy(ones_v, acc.at[row_s.at[j]], add=True)
    return carry

  lax.fori_loop(0, CCH, body, 0)
  plsc.subcore_barrier()
  pltpu.sync_copy(acc.at[pl.ds(s * APT, APT)], cnt_h.at[c, s])


def _sc_count(rowc, z, ones128):
  mesh = plsc.VectorSubcoreMesh(core_axis_name="c", subcore_axis_name="s",
                                num_cores=NC, num_subcores=NS)
  fn = pl.kernel(
      _sc_cnt_body,
      out_type=jax.ShapeDtypeStruct((NC, NS, APT, HALF), jnp.float32),
      mesh=mesh,
      scratch_types=[
          pltpu.VMEM((CCH, CH), jnp.int32),
          pltpu.VMEM((CH, HALF), jnp.float32),
          pltpu.VMEM_SHARED((NACC, HALF), jnp.float32),
          pltpu.SemaphoreType.DMA,
      ],
  )
  return fn(rowc, z, ones128)


def _tc_body(relu, stacked, h2_ref, agg2_ref, cnt_ref, ws_ref, wn_ref,
             b_ref, o_ref):
  cnt = cnt_ref[0][:, 0:1] + cnt_ref[1][:, 0:1]
  inv = 1.0 / jnp.maximum(cnt, 1.0)
  dn = (((1,), (0,)), ((), ()))
  r = lax.dot_general(h2_ref[0], ws_ref[0:HALF, :], dn,
                      preferred_element_type=jnp.float32)
  r = r + lax.dot_general(h2_ref[1], ws_ref[HALF:, :], dn,
                          preferred_element_type=jnp.float32)
  r = r + lax.dot_general(agg2_ref[0] * inv, wn_ref[0:HALF, :], dn,
                          preferred_element_type=jnp.float32)
  r = r + lax.dot_general(agg2_ref[1] * inv, wn_ref[HALF:, :], dn,
                          preferred_element_type=jnp.float32)
  r = r + b_ref[...]
  if relu:
    r = jnp.maximum(r, 0.0)
  if stacked:
    o_ref[0] = r[:, :HALF]
    o_ref[1] = r[:, HALF:]
  else:
    o_ref[...] = r


def _tc_layer(h2, aggp, cntp, W_self, W_neigh, bias, relu, stacked):
  Bm = 2000
  grid = (N // Bm,)
  if stacked:
    out_shape = jax.ShapeDtypeStruct((NC, N, HALF), jnp.float32)
    out_spec = pl.BlockSpec((NC, Bm, HALF), lambda m: (0, m, 0))
  else:
    out_shape = jax.ShapeDtypeStruct((N, D), jnp.float32)
    out_spec = pl.BlockSpec((Bm, D), lambda m: (m, 0))
  return pl.pallas_call(
      functools.partial(_tc_body, relu, stacked),
      grid=grid,
      in_specs=[
          pl.BlockSpec((NC, Bm, HALF), lambda m: (0, m, 0)),
          pl.BlockSpec((NC, Bm, HALF), lambda m: (0, m, 0)),
          pl.BlockSpec((NC, Bm, HALF), lambda m: (0, m, 0)),
          pl.BlockSpec((D, D), lambda m: (0, 0)),
          pl.BlockSpec((D, D), lambda m: (0, 0)),
          pl.BlockSpec((1, D), lambda m: (0, 0)),
      ],
      out_specs=out_spec,
      out_shape=out_shape,
  )(h2, aggp, cntp, W_self, W_neigh, bias)


def kernel(x, edge_index, edge_weight, W_self_0, W_neigh_0, b_0,
           W_self_1, W_neigh_1, b_1):
  row = edge_index[0]
  col = edge_index[1]
  pad = EPAD - E
  colp = jnp.concatenate([col, jnp.zeros((pad,), col.dtype)])
  rowp = jnp.concatenate([row, jnp.full((pad,), N, row.dtype)])
  ewp = jnp.concatenate([edge_weight, jnp.zeros((pad,), jnp.float32)])
  x2 = jnp.stack([x[:, :HALF], x[:, HALF:]])
  colr = colp.reshape(NS, NGRP, G, CH)
  col2 = jnp.stack([colr, colr + N])
  row_r = rowp.reshape(NS, NGRP, G, CH)
  ew_r = ewp.reshape(NS, NGRP, G * CH)
  rowc = rowp.reshape(NC, NS, CCH, CH)
  z = jnp.zeros((APT, HALF), jnp.float32)
  ones128 = jnp.ones((CH, HALF), jnp.float32)

  cntp = _sc_count(rowc, z, ones128).reshape(NC, NACC, HALF)
  agg0 = _sc_aggregate(x2.reshape(NC * N, HALF), col2, row_r, ew_r,
                       z).reshape(NC, NACC, HALF)
  h2 = _tc_layer(x2, agg0, cntp, W_self_0, W_neigh_0,
                 b_0.reshape(1, D), relu=True, stacked=True)
  agg1 = _sc_aggregate(h2.reshape(NC * N, HALF), col2, row_r, ew_r,
                       z).reshape(NC, NACC, HALF)
  return _tc_layer(h2, agg1, cntp, W_self_1, W_neigh_1,
                   b_1.reshape(1, D), relu=False, stacked=False)


# speedup vs baseline: 3.4063x; 1.0180x over previous
import functools

import jax
import jax.numpy as jnp
from jax import lax
from jax.experimental import pallas as pl
from jax.experimental.pallas import tpu as pltpu
from jax.experimental.pallas import tpu_sc as plsc

N = 10000
E = 160000
D = 256
HALF = 128
NC = 2
NS = 16
LANES = 16
CH = 32
G = 8
NGRP = 40
NCHUNK = NGRP * G
EPT = NCHUNK * CH
EPAD = NS * EPT
NACC = 10240
APT = NACC // NS
CCH = EPAD // NC // NS // CH


def _sc_agg_body(tab_h, col_h, row_h, ew_h, z_h, agg_h,
                 col_s, row_s, ew_s, rows_v, acc,
                 semg0, semg1, semg2, semg3,
                 semc0, semc1, semc2, semc3, semst):
  c = lax.axis_index("c")
  s = lax.axis_index("s")
  sems = (semg0, semg1, semg2, semg3)
  csems = (semc0, semc1, semc2, semc3)

  pltpu.sync_copy(z_h, acc.at[pl.ds(s * APT, APT)])

  def stage_start(g, gb):
    pltpu.async_copy(col_h.at[c, s, g], col_s.at[gb], semst)
    pltpu.async_copy(row_h.at[s, g], row_s.at[gb], semst)
    pltpu.async_copy(ew_h.at[s, g], ew_s.at[gb], semst)

  def stage_wait(g, gb):
    pltpu.make_async_copy(col_h.at[c, s, g], col_s.at[gb], semst).wait()
    pltpu.make_async_copy(row_h.at[s, g], row_s.at[gb], semst).wait()
    pltpu.make_async_copy(ew_h.at[s, g], ew_s.at[gb], semst).wait()

  def issue_gather(gb, k):
    pltpu.async_copy(tab_h.at[col_s.at[gb, k]], rows_v.at[k % 4],
                     sems[k % 4])

  def wait_gather(gb, k):
    pltpu.make_async_copy(tab_h.at[col_s.at[gb, k]], rows_v.at[k % 4],
                          sems[k % 4]).wait()

  def issue_scatter(gb, k):
    pltpu.async_copy(rows_v.at[k % 4], acc.at[row_s.at[gb, k]],
                     csems[k % 4], add=True)

  def wait_scatter(gb, k):
    pltpu.make_async_copy(rows_v.at[k % 4], acc.at[row_s.at[gb, k]],
                          csems[k % 4]).wait()

  def scale(gb, k):
    b = k % 4
    off = k * CH

    def grp(i, carry):
      ew16 = ew_s[gb, pl.ds(off + i * LANES, LANES)]
      for e in range(LANES):
        wv = jnp.full((LANES,), ew16[e], jnp.float32)
        ri = i * LANES + e
        for f in range(HALF // LANES):
          sl = pl.ds(f * LANES, LANES)
          rows_v[b, ri, sl] = rows_v[b, ri, sl] * wv
      return carry

    lax.fori_loop(0, CH // LANES, grp, 0)

  plsc.subcore_barrier()

  stage_start(0, 0)
  stage_wait(0, 0)
  stage_start(1, 1)
  issue_gather(0, 0)
  issue_gather(0, 1)

  def group_body(g, carry):
    gb = lax.rem(g, 2)
    ngb = lax.rem(g + 1, 2)
    for k in range(G):
      wait_gather(gb, k)
      scale(gb, k)
      issue_scatter(gb, k)
      if k >= 2:
        wait_scatter(gb, k - 2)
      else:
        @pl.when(g > 0)
        def _():
          wait_scatter(ngb, G - 2 + k)
      if k <= G - 3:
        issue_gather(gb, k + 2)
      if k == G - 3:
        @pl.when(g + 1 < NGRP)
        def _():
          stage_wait(g + 1, ngb)
      if k == G - 2:
        @pl.when(g + 1 < NGRP)
        def _():
          issue_gather(ngb, 0)
      if k == G - 1:
        @pl.when(g + 1 < NGRP)
        def _():
          issue_gather(ngb, 1)

        @pl.when(g + 2 < NGRP)
        def _():
          stage_start(g + 2, gb)
    return carry

  lax.fori_loop(0, NGRP, group_body, 0)

  lgb = (NGRP - 1) % 2
  wait_scatter(lgb, G - 2)
  wait_scatter(lgb, G - 1)
  plsc.subcore_barrier()
  pltpu.sync_copy(acc.at[pl.ds(s * APT, APT)], agg_h.at[c, s])


def _sc_aggregate(tab, col2, row_r, ew_r, z):
  mesh = plsc.VectorSubcoreMesh(core_axis_name="c", subcore_axis_name="s",
                                num_cores=NC, num_subcores=NS)
  fn = pl.kernel(
      _sc_agg_body,
      out_type=jax.ShapeDtypeStruct((NC, NS, APT, HALF), jnp.float32),
      mesh=mesh,
      scratch_types=[
          pltpu.VMEM((2, G, CH), jnp.int32),
          pltpu.VMEM((2, G, CH), jnp.int32),
          pltpu.VMEM((2, G * CH), jnp.float32),
          pltpu.VMEM((4, CH, HALF), jnp.float32),
          pltpu.VMEM_SHARED((NACC, HALF), jnp.float32),
      ] + [pltpu.SemaphoreType.DMA] * 9,
  )
  return fn(tab, col2, row_r, ew_r, z)


def _sc_cnt_body(rowc_h, z_h, ones_h, cnt_h, row_s, ones_v, acc, sem):
  c = lax.axis_index("c")
  s = lax.axis_index("s")
  pltpu.sync_copy(rowc_h.at[c, s], row_s)
  pltpu.sync_copy(ones_h, ones_v)
  pltpu.sync_copy(z_h, acc.at[pl.ds(s * APT, APT)])
  plsc.subcore_barrier()

  def body(j, carry):
    pltpu.sync_copy(ones_v, acc.at[row_s.at[j]], add=True)
    return carry

  lax.fori_loop(0, CCH, body, 0)
  plsc.subcore_barrier()
  pltpu.sync_copy(acc.at[pl.ds(s * APT, APT)], cnt_h.at[c, s])


def _sc_count(rowc, z, ones128):
  mesh = plsc.VectorSubcoreMesh(core_axis_name="c", subcore_axis_name="s",
                                num_cores=NC, num_subcores=NS)
  fn = pl.kernel(
      _sc_cnt_body,
      out_type=jax.ShapeDtypeStruct((NC, NS, APT, HALF), jnp.float32),
      mesh=mesh,
      scratch_types=[
          pltpu.VMEM((CCH, CH), jnp.int32),
          pltpu.VMEM((CH, HALF), jnp.float32),
          pltpu.VMEM_SHARED((NACC, HALF), jnp.float32),
          pltpu.SemaphoreType.DMA,
      ],
  )
  return fn(rowc, z, ones128)


def _tc_body(relu, stacked, h2_ref, agg2_ref, cnt_ref, ws_ref, wn_ref,
             b_ref, o_ref):
  cnt = cnt_ref[0][:, 0:1] + cnt_ref[1][:, 0:1]
  inv = 1.0 / jnp.maximum(cnt, 1.0)
  dn = (((1,), (0,)), ((), ()))
  r = lax.dot_general(h2_ref[0], ws_ref[0:HALF, :], dn,
                      preferred_element_type=jnp.float32)
  r = r + lax.dot_general(h2_ref[1], ws_ref[HALF:, :], dn,
                          preferred_element_type=jnp.float32)
  r = r + lax.dot_general(agg2_ref[0] * inv, wn_ref[0:HALF, :], dn,
                          preferred_element_type=jnp.float32)
  r = r + lax.dot_general(agg2_ref[1] * inv, wn_ref[HALF:, :], dn,
                          preferred_element_type=jnp.float32)
  r = r + b_ref[...]
  if relu:
    r = jnp.maximum(r, 0.0)
  if stacked:
    o_ref[0] = r[:, :HALF]
    o_ref[1] = r[:, HALF:]
  else:
    o_ref[...] = r


def _tc_layer(h2, aggp, cntp, W_self, W_neigh, bias, relu, stacked):
  Bm = 2000
  grid = (N // Bm,)
  if stacked:
    out_shape = jax.ShapeDtypeStruct((NC, N, HALF), jnp.float32)
    out_spec = pl.BlockSpec((NC, Bm, HALF), lambda m: (0, m, 0))
  else:
    out_shape = jax.ShapeDtypeStruct((N, D), jnp.float32)
    out_spec = pl.BlockSpec((Bm, D), lambda m: (m, 0))
  return pl.pallas_call(
      functools.partial(_tc_body, relu, stacked),
      grid=grid,
      in_specs=[
          pl.BlockSpec((NC, Bm, HALF), lambda m: (0, m, 0)),
          pl.BlockSpec((NC, Bm, HALF), lambda m: (0, m, 0)),
          pl.BlockSpec((NC, Bm, HALF), lambda m: (0, m, 0)),
          pl.BlockSpec((D, D), lambda m: (0, 0)),
          pl.BlockSpec((D, D), lambda m: (0, 0)),
          pl.BlockSpec((1, D), lambda m: (0, 0)),
      ],
      out_specs=out_spec,
      out_shape=out_shape,
  )(h2, aggp, cntp, W_self, W_neigh, bias)


def kernel(x, edge_index, edge_weight, W_self_0, W_neigh_0, b_0,
           W_self_1, W_neigh_1, b_1):
  row = edge_index[0]
  col = edge_index[1]
  pad = EPAD - E
  colp = jnp.concatenate([col, jnp.zeros((pad,), col.dtype)])
  rowp = jnp.concatenate([row, jnp.full((pad,), N, row.dtype)])
  ewp = jnp.concatenate([edge_weight, jnp.zeros((pad,), jnp.float32)])
  x2 = jnp.stack([x[:, :HALF], x[:, HALF:]])
  colr = colp.reshape(NS, NGRP, G, CH)
  col2 = jnp.stack([colr, colr + N])
  row_r = rowp.reshape(NS, NGRP, G, CH)
  ew_r = ewp.reshape(NS, NGRP, G * CH)
  rowc = rowp.reshape(NC, NS, CCH, CH)
  z = jnp.zeros((APT, HALF), jnp.float32)
  ones128 = jnp.ones((CH, HALF), jnp.float32)

  cntp = _sc_count(rowc, z, ones128).reshape(NC, NACC, HALF)
  agg0 = _sc_aggregate(x2.reshape(NC * N, HALF), col2, row_r, ew_r,
                       z).reshape(NC, NACC, HALF)
  h2 = _tc_layer(x2, agg0, cntp, W_self_0, W_neigh_0,
                 b_0.reshape(1, D), relu=True, stacked=True)
  agg1 = _sc_aggregate(h2.reshape(NC * N, HALF), col2, row_r, ew_r,
                       z).reshape(NC, NACC, HALF)
  return _tc_layer(h2, agg1, cntp, W_self_1, W_neigh_1,
                   b_1.reshape(1, D), relu=False, stacked=False)


# speedup vs baseline: 3.5461x; 1.0410x over previous
import functools

import jax
import jax.numpy as jnp
from jax import lax
from jax.experimental import pallas as pl
from jax.experimental.pallas import tpu as pltpu
from jax.experimental.pallas import tpu_sc as plsc

N = 10000
E = 160000
D = 256
HALF = 128
NC = 2
NS = 16
LANES = 16
CH = 64
G = 4
NGRP = 40
NCHUNK = NGRP * G
EPT = NCHUNK * CH
EPAD = NS * EPT
NACC = 10240
APT = NACC // NS
CCH = EPAD // NC // NS // CH


def _sc_agg_body(tab_h, col_h, row_h, ew_h, z_h, agg_h,
                 col_s, row_s, ew_s, rows_v, acc,
                 semg0, semg1, semg2, semg3,
                 semc0, semc1, semc2, semc3, semst):
  c = lax.axis_index("c")
  s = lax.axis_index("s")
  sems = (semg0, semg1, semg2, semg3)
  csems = (semc0, semc1, semc2, semc3)

  pltpu.sync_copy(z_h, acc.at[pl.ds(s * APT, APT)])

  def stage_start(g, gb):
    pltpu.async_copy(col_h.at[c, s, g], col_s.at[gb], semst)
    pltpu.async_copy(row_h.at[s, g], row_s.at[gb], semst)
    pltpu.async_copy(ew_h.at[s, g], ew_s.at[gb], semst)

  def stage_wait(g, gb):
    pltpu.make_async_copy(col_h.at[c, s, g], col_s.at[gb], semst).wait()
    pltpu.make_async_copy(row_h.at[s, g], row_s.at[gb], semst).wait()
    pltpu.make_async_copy(ew_h.at[s, g], ew_s.at[gb], semst).wait()

  def issue_gather(gb, k):
    pltpu.async_copy(tab_h.at[col_s.at[gb, k]], rows_v.at[k % 4],
                     sems[k % 4])

  def wait_gather(gb, k):
    pltpu.make_async_copy(tab_h.at[col_s.at[gb, k]], rows_v.at[k % 4],
                          sems[k % 4]).wait()

  def issue_scatter(gb, k):
    pltpu.async_copy(rows_v.at[k % 4], acc.at[row_s.at[gb, k]],
                     csems[k % 4], add=True)

  def wait_scatter(gb, k):
    pltpu.make_async_copy(rows_v.at[k % 4], acc.at[row_s.at[gb, k]],
                          csems[k % 4]).wait()

  def scale(gb, k):
    b = k % 4
    off = k * CH

    def grp(i, carry):
      ew16 = ew_s[gb, pl.ds(off + i * LANES, LANES)]
      for e in range(LANES):
        wv = jnp.full((LANES,), ew16[e], jnp.float32)
        ri = i * LANES + e
        for f in range(HALF // LANES):
          sl = pl.ds(f * LANES, LANES)
          rows_v[b, ri, sl] = rows_v[b, ri, sl] * wv
      return carry

    lax.fori_loop(0, CH // LANES, grp, 0, unroll=2)

  plsc.subcore_barrier()

  stage_start(0, 0)
  stage_wait(0, 0)
  stage_start(1, 1)
  issue_gather(0, 0)
  issue_gather(0, 1)

  def group_body(g, carry):
    gb = lax.rem(g, 2)
    ngb = lax.rem(g + 1, 2)
    for k in range(G):
      wait_gather(gb, k)
      scale(gb, k)
      issue_scatter(gb, k)
      if k >= 2:
        wait_scatter(gb, k - 2)
      else:
        @pl.when(g > 0)
        def _():
          wait_scatter(ngb, G - 2 + k)
      if k <= G - 3:
        issue_gather(gb, k + 2)
      if k == G - 3:
        @pl.when(g + 1 < NGRP)
        def _():
          stage_wait(g + 1, ngb)
      if k == G - 2:
        @pl.when(g + 1 < NGRP)
        def _():
          issue_gather(ngb, 0)
      if k == G - 1:
        @pl.when(g + 1 < NGRP)
        def _():
          issue_gather(ngb, 1)

        @pl.when(g + 2 < NGRP)
        def _():
          stage_start(g + 2, gb)
    return carry

  lax.fori_loop(0, NGRP, group_body, 0)

  lgb = (NGRP - 1) % 2
  wait_scatter(lgb, G - 2)
  wait_scatter(lgb, G - 1)
  plsc.subcore_barrier()
  pltpu.sync_copy(acc.at[pl.ds(s * APT, APT)], agg_h.at[c, s])


def _sc_aggregate(tab, col2, row_r, ew_r, z):
  mesh = plsc.VectorSubcoreMesh(core_axis_name="c", subcore_axis_name="s",
                                num_cores=NC, num_subcores=NS)
  fn = pl.kernel(
      _sc_agg_body,
      out_type=jax.ShapeDtypeStruct((NC, NS, APT, HALF), jnp.float32),
      mesh=mesh,
      scratch_types=[
          pltpu.VMEM((2, G, CH), jnp.int32),
          pltpu.VMEM((2, G, CH), jnp.int32),
          pltpu.VMEM((2, G * CH), jnp.float32),
          pltpu.VMEM((4, CH, HALF), jnp.float32),
          pltpu.VMEM_SHARED((NACC, HALF), jnp.float32),
      ] + [pltpu.SemaphoreType.DMA] * 9,
  )
  return fn(tab, col2, row_r, ew_r, z)


def _sc_cnt_body(rowc_h, z_h, ones_h, cnt_h, row_s, ones_v, acc, sem):
  c = lax.axis_index("c")
  s = lax.axis_index("s")
  pltpu.sync_copy(rowc_h.at[c, s], row_s)
  pltpu.sync_copy(ones_h, ones_v)
  pltpu.sync_copy(z_h, acc.at[pl.ds(s * APT, APT)])
  plsc.subcore_barrier()

  def body(j, carry):
    pltpu.sync_copy(ones_v, acc.at[row_s.at[j]], add=True)
    return carry

  lax.fori_loop(0, CCH, body, 0)
  plsc.subcore_barrier()
  pltpu.sync_copy(acc.at[pl.ds(s * APT, APT)], cnt_h.at[c, s])


def _sc_count(rowc, z, ones128):
  mesh = plsc.VectorSubcoreMesh(core_axis_name="c", subcore_axis_name="s",
                                num_cores=NC, num_subcores=NS)
  fn = pl.kernel(
      _sc_cnt_body,
      out_type=jax.ShapeDtypeStruct((NC, NS, APT, HALF), jnp.float32),
      mesh=mesh,
      scratch_types=[
          pltpu.VMEM((CCH, CH), jnp.int32),
          pltpu.VMEM((CH, HALF), jnp.float32),
          pltpu.VMEM_SHARED((NACC, HALF), jnp.float32),
          pltpu.SemaphoreType.DMA,
      ],
  )
  return fn(rowc, z, ones128)


def _tc_body(relu, stacked, h2_ref, agg2_ref, cnt_ref, ws_ref, wn_ref,
             b_ref, o_ref):
  cnt = cnt_ref[0][:, 0:1] + cnt_ref[1][:, 0:1]
  inv = 1.0 / jnp.maximum(cnt, 1.0)
  dn = (((1,), (0,)), ((), ()))
  r = lax.dot_general(h2_ref[0], ws_ref[0:HALF, :], dn,
                      preferred_element_type=jnp.float32)
  r = r + lax.dot_general(h2_ref[1], ws_ref[HALF:, :], dn,
                          preferred_element_type=jnp.float32)
  r = r + lax.dot_general(agg2_ref[0] * inv, wn_ref[0:HALF, :], dn,
                          preferred_element_type=jnp.float32)
  r = r + lax.dot_general(agg2_ref[1] * inv, wn_ref[HALF:, :], dn,
                          preferred_element_type=jnp.float32)
  r = r + b_ref[...]
  if relu:
    r = jnp.maximum(r, 0.0)
  if stacked:
    o_ref[0] = r[:, :HALF]
    o_ref[1] = r[:, HALF:]
  else:
    o_ref[...] = r


def _tc_layer(h2, aggp, cntp, W_self, W_neigh, bias, relu, stacked):
  Bm = 2000
  grid = (N // Bm,)
  if stacked:
    out_shape = jax.ShapeDtypeStruct((NC, N, HALF), jnp.float32)
    out_spec = pl.BlockSpec((NC, Bm, HALF), lambda m: (0, m, 0))
  else:
    out_shape = jax.ShapeDtypeStruct((N, D), jnp.float32)
    out_spec = pl.BlockSpec((Bm, D), lambda m: (m, 0))
  return pl.pallas_call(
      functools.partial(_tc_body, relu, stacked),
      grid=grid,
      in_specs=[
          pl.BlockSpec((NC, Bm, HALF), lambda m: (0, m, 0)),
          pl.BlockSpec((NC, Bm, HALF), lambda m: (0, m, 0)),
          pl.BlockSpec((NC, Bm, HALF), lambda m: (0, m, 0)),
          pl.BlockSpec((D, D), lambda m: (0, 0)),
          pl.BlockSpec((D, D), lambda m: (0, 0)),
          pl.BlockSpec((1, D), lambda m: (0, 0)),
      ],
      out_specs=out_spec,
      out_shape=out_shape,
  )(h2, aggp, cntp, W_self, W_neigh, bias)


def kernel(x, edge_index, edge_weight, W_self_0, W_neigh_0, b_0,
           W_self_1, W_neigh_1, b_1):
  row = edge_index[0]
  col = edge_index[1]
  pad = EPAD - E
  colp = jnp.concatenate([col, jnp.zeros((pad,), col.dtype)])
  rowp = jnp.concatenate([row, jnp.full((pad,), N, row.dtype)])
  ewp = jnp.concatenate([edge_weight, jnp.zeros((pad,), jnp.float32)])
  x2 = jnp.stack([x[:, :HALF], x[:, HALF:]])
  colr = colp.reshape(NS, NGRP, G, CH)
  col2 = jnp.stack([colr, colr + N])
  row_r = rowp.reshape(NS, NGRP, G, CH)
  ew_r = ewp.reshape(NS, NGRP, G * CH)
  rowc = rowp.reshape(NC, NS, CCH, CH)
  z = jnp.zeros((APT, HALF), jnp.float32)
  ones128 = jnp.ones((CH, HALF), jnp.float32)

  cntp = _sc_count(rowc, z, ones128).reshape(NC, NACC, HALF)
  agg0 = _sc_aggregate(x2.reshape(NC * N, HALF), col2, row_r, ew_r,
                       z).reshape(NC, NACC, HALF)
  h2 = _tc_layer(x2, agg0, cntp, W_self_0, W_neigh_0,
                 b_0.reshape(1, D), relu=True, stacked=True)
  agg1 = _sc_aggregate(h2.reshape(NC * N, HALF), col2, row_r, ew_r,
                       z).reshape(NC, NACC, HALF)
  return _tc_layer(h2, agg1, cntp, W_self_1, W_neigh_1,
                   b_1.reshape(1, D), relu=False, stacked=False)
